# baseline (speedup 1.0000x reference)
import functools

import jax
import jax.numpy as jnp
from jax import lax
from jax.experimental import pallas as pl
from jax.experimental.pallas import tpu as pltpu
from jax.experimental.pallas import tpu_sc as plsc

N = 10000
E = 320000
DIN = 128
HID = 16
NCLS = 40

NCORE = 2
NSUB = 16
NW = NCORE * NSUB
B = 128
K = 80
E_PAD = NW * B * K
NPAD = 10112
RPS = NPAD // NSUB
DEGW = 16
NB = 8

_SC_PARAMS = pltpu.CompilerParams(use_tc_tiling_on_sc=False)


def _sc_agg():
    D = HID
    mesh = plsc.VectorSubcoreMesh(core_axis_name="c", subcore_axis_name="s")

    @functools.partial(
        pl.kernel,
        out_type=jax.ShapeDtypeStruct((NCORE, NPAD, D), jnp.float32),
        mesh=mesh,
        scratch_types=[
            pltpu.VMEM((K, B), jnp.int32),
            pltpu.VMEM((K, B), jnp.int32),
            pltpu.VMEM((NB, B, D), jnp.float32),
            pltpu.VMEM_SHARED((NPAD, D), jnp.float32),
        ] + [pltpu.SemaphoreType.DMA] * (2 * NB),
        compiler_params=_SC_PARAMS,
    )
    def agg(hs_hbm, src_hbm, dst_hbm, zeros_hbm, out_hbm,
            src_v, dst_v, rows_v, acc_sh, *sems):
        gsem = sems[:NB]
        ssem = sems[NB:]
        c = lax.axis_index("c")
        s = lax.axis_index("s")
        w = s * NCORE + c
        pltpu.sync_copy(src_hbm.at[pl.ds(w * K, K)], src_v)
        pltpu.sync_copy(dst_hbm.at[pl.ds(w * K, K)], dst_v)
        pltpu.sync_copy(zeros_hbm, acc_sh.at[pl.ds(s * RPS, RPS)])
        plsc.subcore_barrier()

        pltpu.async_copy(hs_hbm.at[src_v.at[0]], rows_v.at[0], gsem[0])

        def outer(t, carry):
            for b in range(NB):
                j = t * NB + b
                bn = (b + 1) % NB
                jn = j + 1
                pltpu.make_async_copy(
                    hs_hbm.at[src_v.at[j]], rows_v.at[b], gsem[b]).wait()
                pltpu.async_copy(rows_v.at[b],
                                 acc_sh.at[dst_v.at[j]], ssem[b], add=True)

                @pl.when(jnp.logical_and(jn >= NB, jn < K))
                def _():
                    pltpu.make_async_copy(
                        rows_v.at[bn],
                        acc_sh.at[dst_v.at[jn - NB]], ssem[bn]).wait()

                @pl.when(jn < K)
                def _():
                    pltpu.async_copy(
                        hs_hbm.at[src_v.at[jn]], rows_v.at[bn], gsem[bn])
            return carry

        lax.fori_loop(0, K // NB, outer, 0)
        for cch in range(K - NB, K):
            pltpu.make_async_copy(
                rows_v.at[cch % NB],
                acc_sh.at[dst_v.at[cch]], ssem[cch % NB]).wait()
        plsc.subcore_barrier()
        pltpu.sync_copy(acc_sh.at[pl.ds(s * RPS, RPS)],
                        out_hbm.at[c, pl.ds(s * RPS, RPS)])

    return agg


def _sc_deg():
    mesh = plsc.VectorSubcoreMesh(core_axis_name="c", subcore_axis_name="s")

    @functools.partial(
        pl.kernel,
        out_type=jax.ShapeDtypeStruct((NCORE, NPAD, DEGW), jnp.float32),
        mesh=mesh,
        scratch_types=[
            pltpu.VMEM((K, B), jnp.int32),
            pltpu.VMEM((B, DEGW), jnp.float32),
            pltpu.VMEM_SHARED((NPAD, DEGW), jnp.float32),
        ],
        compiler_params=_SC_PARAMS,
    )
    def deg(dst_hbm, ones_hbm, zeros_hbm, out_hbm, dst_v, ones_v, acc_sh):
        c = lax.axis_index("c")
        s = lax.axis_index("s")
        w = s * NCORE + c
        pltpu.sync_copy(dst_hbm.at[pl.ds(w * K, K)], dst_v)
        pltpu.sync_copy(ones_hbm, ones_v)
        pltpu.sync_copy(zeros_hbm, acc_sh.at[pl.ds(s * RPS, RPS)])
        plsc.subcore_barrier()

        def step(j, carry):
            pltpu.sync_copy(ones_v, acc_sh.at[dst_v.at[j]], add=True)
            return carry

        lax.fori_loop(0, K, step, 0)
        plsc.subcore_barrier()
        pltpu.sync_copy(acc_sh.at[pl.ds(s * RPS, RPS)],
                        out_hbm.at[c, pl.ds(s * RPS, RPS)])

    return deg


def _tc_l1(x, W1, degp):
    def body(x_ref, w_ref, d_ref, h_ref, hs_ref, dinv_ref):
        h = jnp.dot(x_ref[...], w_ref[...],
                    preferred_element_type=jnp.float32)
        deg = d_ref[0, :, :1] + d_ref[1, :, :1] + 1.0
        dinv = lax.rsqrt(deg)
        h_ref[...] = h
        hs_ref[...] = h * dinv
        dinv_ref[...] = dinv

    return pl.pallas_call(
        body,
        out_shape=[
            jax.ShapeDtypeStruct((N, HID), jnp.float32),
            jax.ShapeDtypeStruct((N, HID), jnp.float32),
            jax.ShapeDtypeStruct((N, 1), jnp.float32),
        ],
    )(x, W1, degp)


def _tc_relu(p, h1, dinv, b1):
    def body(p_ref, h1_ref, dinv_ref, b1_ref, z_ref, zs_ref):
        dinv = dinv_ref[...]
        z = dinv * (p_ref[0] + p_ref[1]) + (dinv * dinv) * h1_ref[...] \
            + b1_ref[...]
        z = jnp.maximum(z, 0.0)
        z_ref[...] = z
        zs_ref[...] = z * dinv

    return pl.pallas_call(
        body,
        out_shape=[
            jax.ShapeDtypeStruct((N, HID), jnp.float32),
            jax.ShapeDtypeStruct((N, HID), jnp.float32),
        ],
    )(p, h1, dinv, b1.reshape(1, HID))


def _tc_out(q, z1, dinv, W2, b2):
    def body(q_ref, z_ref, dinv_ref, w2_ref, b2_ref, out_ref):
        dinv = dinv_ref[...]
        agg = dinv * (q_ref[0] + q_ref[1]) + (dinv * dinv) * z_ref[...]
        out_ref[...] = jnp.dot(agg, w2_ref[...],
                               preferred_element_type=jnp.float32) \
            + b2_ref[...]

    return pl.pallas_call(
        body,
        out_shape=jax.ShapeDtypeStruct((N, NCLS), jnp.float32),
    )(q, z1, dinv, W2, b2.reshape(1, NCLS))


def kernel(x, edge_index, W1, b1, W2, b2):
    src = edge_index[0]
    dst = edge_index[1]
    pad = E_PAD - E
    src_p = jnp.concatenate(
        [src, jnp.zeros((pad,), jnp.int32)]).reshape(NW * K, B)
    dst_p = jnp.concatenate(
        [dst, jnp.full((pad,), N, jnp.int32)]).reshape(NW * K, B)

    ones_deg = jnp.ones((B, DEGW), jnp.float32)
    zeros_deg = jnp.zeros((RPS, DEGW), jnp.float32)
    zeros_h = jnp.zeros((RPS, HID), jnp.float32)

    agg = _sc_agg()
    degp = _sc_deg()(dst_p, ones_deg, zeros_deg)
    h1, hs1, dinv = _tc_l1(x, W1, degp[:, :N])
    p = agg(hs1, src_p, dst_p, zeros_h)
    z1, zs1 = _tc_relu(p[:, :N], h1, dinv, b1)
    q = agg(zs1, src_p, dst_p, zeros_h)
    return _tc_out(q[:, :N], z1, dinv, W2, b2)


# speedup vs baseline: 1.6754x; 1.6754x over previous
import functools

import jax
import jax.numpy as jnp
from jax import lax
from jax.experimental import pallas as pl
from jax.experimental.pallas import tpu as pltpu
from jax.experimental.pallas import tpu_sc as plsc

N = 10000
E = 320000
DIN = 128
HID = 16
NCLS = 40

NCORE = 2
NSUB = 16
NW = NCORE * NSUB
B = 128
K = 80
E_PAD = NW * B * K
NPAD = 10112
RPS = NPAD // NSUB
DEGW = 16
NB = 8

_SC_PARAMS = pltpu.CompilerParams(use_tc_tiling_on_sc=False)


def _sc_agg():
    D = HID
    mesh = plsc.VectorSubcoreMesh(core_axis_name="c", subcore_axis_name="s")

    @functools.partial(
        pl.kernel,
        out_type=jax.ShapeDtypeStruct((NCORE, NPAD, D), jnp.float32),
        mesh=mesh,
        scratch_types=[
            pltpu.VMEM((K, B), jnp.int32),
            pltpu.VMEM((K, B), jnp.int32),
            pltpu.VMEM((NB, B, D), jnp.float32),
            pltpu.VMEM_SHARED((NPAD, D), jnp.float32),
            pltpu.VMEM_SHARED((NPAD, D), jnp.float32),
        ] + [pltpu.SemaphoreType.DMA] * (2 * NB),
        compiler_params=_SC_PARAMS,
    )
    def agg(hs_hbm, src_hbm, dst_hbm, zeros_hbm, out_hbm,
            src_v, dst_v, rows_v, acc_sh, tab_sh, *sems):
        gsem = sems[:NB]
        ssem = sems[NB:]
        c = lax.axis_index("c")
        s = lax.axis_index("s")
        w = s * NCORE + c
        pltpu.sync_copy(src_hbm.at[pl.ds(w * K, K)], src_v)
        pltpu.sync_copy(dst_hbm.at[pl.ds(w * K, K)], dst_v)
        pltpu.sync_copy(zeros_hbm, acc_sh.at[pl.ds(s * RPS, RPS)])
        pltpu.sync_copy(hs_hbm.at[pl.ds(s * RPS, RPS)],
                        tab_sh.at[pl.ds(s * RPS, RPS)])
        plsc.subcore_barrier()

        pltpu.async_copy(tab_sh.at[src_v.at[0]], rows_v.at[0], gsem[0])

        def outer(t, carry):
            for b in range(NB):
                j = t * NB + b
                bn = (b + 1) % NB
                jn = j + 1
                pltpu.make_async_copy(
                    tab_sh.at[src_v.at[j]], rows_v.at[b], gsem[b]).wait()
                pltpu.async_copy(rows_v.at[b],
                                 acc_sh.at[dst_v.at[j]], ssem[b], add=True)

                @pl.when(jnp.logical_and(jn >= NB, jn < K))
                def _():
                    pltpu.make_async_copy(
                        rows_v.at[bn],
                        acc_sh.at[dst_v.at[jn - NB]], ssem[bn]).wait()

                @pl.when(jn < K)
                def _():
                    pltpu.async_copy(
                        tab_sh.at[src_v.at[jn]], rows_v.at[bn], gsem[bn])
            return carry

        lax.fori_loop(0, K // NB, outer, 0)
        for cch in range(K - NB, K):
            pltpu.make_async_copy(
                rows_v.at[cch % NB],
                acc_sh.at[dst_v.at[cch]], ssem[cch % NB]).wait()
        plsc.subcore_barrier()
        pltpu.sync_copy(acc_sh.at[pl.ds(s * RPS, RPS)],
                        out_hbm.at[c, pl.ds(s * RPS, RPS)])

    return agg


def _sc_deg():
    mesh = plsc.VectorSubcoreMesh(core_axis_name="c", subcore_axis_name="s")

    @functools.partial(
        pl.kernel,
        out_type=jax.ShapeDtypeStruct((NCORE, NPAD, DEGW), jnp.float32),
        mesh=mesh,
        scratch_types=[
            pltpu.VMEM((K, B), jnp.int32),
            pltpu.VMEM((B, DEGW), jnp.float32),
            pltpu.VMEM_SHARED((NPAD, DEGW), jnp.float32),
        ],
        compiler_params=_SC_PARAMS,
    )
    def deg(dst_hbm, ones_hbm, zeros_hbm, out_hbm, dst_v, ones_v, acc_sh):
        c = lax.axis_index("c")
        s = lax.axis_index("s")
        w = s * NCORE + c
        pltpu.sync_copy(dst_hbm.at[pl.ds(w * K, K)], dst_v)
        pltpu.sync_copy(ones_hbm, ones_v)
        pltpu.sync_copy(zeros_hbm, acc_sh.at[pl.ds(s * RPS, RPS)])
        plsc.subcore_barrier()

        def step(j, carry):
            pltpu.sync_copy(ones_v, acc_sh.at[dst_v.at[j]], add=True)
            return carry

        lax.fori_loop(0, K, step, 0)
        plsc.subcore_barrier()
        pltpu.sync_copy(acc_sh.at[pl.ds(s * RPS, RPS)],
                        out_hbm.at[c, pl.ds(s * RPS, RPS)])

    return deg


def _tc_l1(x, W1, degp):
    def body(x_ref, w_ref, d_ref, h_ref, hs_ref, dinv_ref):
        h = jnp.dot(x_ref[...], w_ref[...],
                    preferred_element_type=jnp.float32)
        deg = d_ref[0, :, :1] + d_ref[1, :, :1] + 1.0
        dinv = lax.rsqrt(deg)
        h_ref[...] = h
        hs_ref[...] = h * dinv
        dinv_ref[...] = dinv

    return pl.pallas_call(
        body,
        out_shape=[
            jax.ShapeDtypeStruct((N, HID), jnp.float32),
            jax.ShapeDtypeStruct((N, HID), jnp.float32),
            jax.ShapeDtypeStruct((N, 1), jnp.float32),
        ],
    )(x, W1, degp)


def _tc_relu(p, h1, dinv, b1):
    def body(p_ref, h1_ref, dinv_ref, b1_ref, z_ref, zs_ref):
        dinv = dinv_ref[...]
        z = dinv * (p_ref[0] + p_ref[1]) + (dinv * dinv) * h1_ref[...] \
            + b1_ref[...]
        z = jnp.maximum(z, 0.0)
        z_ref[...] = z
        zs_ref[...] = z * dinv

    return pl.pallas_call(
        body,
        out_shape=[
            jax.ShapeDtypeStruct((N, HID), jnp.float32),
            jax.ShapeDtypeStruct((N, HID), jnp.float32),
        ],
    )(p, h1, dinv, b1.reshape(1, HID))


def _tc_out(q, z1, dinv, W2, b2):
    def body(q_ref, z_ref, dinv_ref, w2_ref, b2_ref, out_ref):
        dinv = dinv_ref[...]
        agg = dinv * (q_ref[0] + q_ref[1]) + (dinv * dinv) * z_ref[...]
        out_ref[...] = jnp.dot(agg, w2_ref[...],
                               preferred_element_type=jnp.float32) \
            + b2_ref[...]

    return pl.pallas_call(
        body,
        out_shape=jax.ShapeDtypeStruct((N, NCLS), jnp.float32),
    )(q, z1, dinv, W2, b2.reshape(1, NCLS))


def kernel(x, edge_index, W1, b1, W2, b2):
    src = edge_index[0]
    dst = edge_index[1]
    pad = E_PAD - E
    src_p = jnp.concatenate(
        [src, jnp.zeros((pad,), jnp.int32)]).reshape(NW * K, B)
    dst_p = jnp.concatenate(
        [dst, jnp.full((pad,), N, jnp.int32)]).reshape(NW * K, B)

    ones_deg = jnp.ones((B, DEGW), jnp.float32)
    zeros_deg = jnp.zeros((RPS, DEGW), jnp.float32)
    zeros_h = jnp.zeros((RPS, HID), jnp.float32)

    agg = _sc_agg()
    degp = _sc_deg()(dst_p, ones_deg, zeros_deg)
    h1, hs1, dinv = _tc_l1(x, W1, degp[:, :N])
    hs1p = jnp.pad(hs1, ((0, NPAD - N), (0, 0)))
    p = agg(hs1p, src_p, dst_p, zeros_h)
    z1, zs1 = _tc_relu(p[:, :N], h1, dinv, b1)
    zs1p = jnp.pad(zs1, ((0, NPAD - N), (0, 0)))
    q = agg(zs1p, src_p, dst_p, zeros_h)
    return _tc_out(q[:, :N], z1, dinv, W2, b2)


# speedup vs baseline: 1.8563x; 1.1080x over previous
import functools

import jax
import jax.numpy as jnp
from jax import lax
from jax.experimental import pallas as pl
from jax.experimental.pallas import tpu as pltpu
from jax.experimental.pallas import tpu_sc as plsc

N = 10000
E = 320000
DIN = 128
HID = 16
NCLS = 40

NCORE = 2
NSUB = 16
NW = NCORE * NSUB
B = 128
K = 80
E_PAD = NW * B * K
NPAD = 10112
RPS = NPAD // NSUB
DEGW = 16
NB = 8

_SC_PARAMS = pltpu.CompilerParams(use_tc_tiling_on_sc=False)


def _sc_agg():
    D = HID
    mesh = plsc.VectorSubcoreMesh(core_axis_name="c", subcore_axis_name="s")

    @functools.partial(
        pl.kernel,
        out_type=jax.ShapeDtypeStruct((NCORE, NPAD, D), jnp.float32),
        mesh=mesh,
        scratch_types=[
            pltpu.VMEM((K, B), jnp.int32),
            pltpu.VMEM((K, B), jnp.int32),
            pltpu.VMEM((NB, B, D), jnp.float32),
            pltpu.VMEM_SHARED((NPAD, D), jnp.float32),
            pltpu.VMEM_SHARED((NPAD, D), jnp.float32),
        ] + [pltpu.SemaphoreType.DMA] * (2 * NB),
        compiler_params=_SC_PARAMS,
    )
    def agg(hs_hbm, src_hbm, dst_hbm, zeros_hbm, out_hbm,
            src_v, dst_v, rows_v, acc_sh, tab_sh, *sems):
        gsem = sems[:NB]
        ssem = sems[NB:]
        c = lax.axis_index("c")
        s = lax.axis_index("s")
        w = s * NCORE + c
        pltpu.sync_copy(src_hbm.at[pl.ds(w * K, K)], src_v)
        pltpu.sync_copy(dst_hbm.at[pl.ds(w * K, K)], dst_v)
        pltpu.sync_copy(zeros_hbm, acc_sh.at[pl.ds(s * RPS, RPS)])
        pltpu.sync_copy(hs_hbm.at[pl.ds(s * RPS, RPS)],
                        tab_sh.at[pl.ds(s * RPS, RPS)])
        plsc.subcore_barrier()

        pltpu.async_copy(tab_sh.at[src_v.at[0]], rows_v.at[0], gsem[0])

        def outer(t, carry):
            for b in range(NB):
                j = t * NB + b
                bn = (b + 1) % NB
                jn = j + 1
                pltpu.make_async_copy(
                    tab_sh.at[src_v.at[j]], rows_v.at[b], gsem[b]).wait()
                pltpu.async_copy(rows_v.at[b],
                                 acc_sh.at[dst_v.at[j]], ssem[b], add=True)

                @pl.when(jnp.logical_and(jn >= NB, jn < K))
                def _():
                    pltpu.make_async_copy(
                        rows_v.at[bn],
                        acc_sh.at[dst_v.at[jn - NB]], ssem[bn]).wait()

                @pl.when(jn < K)
                def _():
                    pltpu.async_copy(
                        tab_sh.at[src_v.at[jn]], rows_v.at[bn], gsem[bn])
            return carry

        lax.fori_loop(0, K // NB, outer, 0)
        for cch in range(K - NB, K):
            pltpu.make_async_copy(
                rows_v.at[cch % NB],
                acc_sh.at[dst_v.at[cch]], ssem[cch % NB]).wait()
        plsc.subcore_barrier()
        pltpu.sync_copy(acc_sh.at[pl.ds(s * RPS, RPS)],
                        out_hbm.at[c, pl.ds(s * RPS, RPS)])

    return agg


def _sc_deg():
    mesh = plsc.VectorSubcoreMesh(core_axis_name="c", subcore_axis_name="s")

    @functools.partial(
        pl.kernel,
        out_type=jax.ShapeDtypeStruct((NCORE, NPAD, DEGW), jnp.float32),
        mesh=mesh,
        scratch_types=[
            pltpu.VMEM((K, B), jnp.int32),
            pltpu.VMEM((B, DEGW), jnp.float32),
            pltpu.VMEM_SHARED((NPAD, DEGW), jnp.float32),
        ],
        compiler_params=_SC_PARAMS,
    )
    def deg(dst_hbm, ones_hbm, zeros_hbm, out_hbm, dst_v, ones_v, acc_sh):
        c = lax.axis_index("c")
        s = lax.axis_index("s")
        w = s * NCORE + c
        pltpu.sync_copy(dst_hbm.at[pl.ds(w * K, K)], dst_v)
        pltpu.sync_copy(ones_hbm, ones_v)
        pltpu.sync_copy(zeros_hbm, acc_sh.at[pl.ds(s * RPS, RPS)])
        plsc.subcore_barrier()

        def step(j, carry):
            pltpu.sync_copy(ones_v, acc_sh.at[dst_v.at[j]], add=True)
            return carry

        lax.fori_loop(0, K, step, 0)
        plsc.subcore_barrier()
        pltpu.sync_copy(acc_sh.at[pl.ds(s * RPS, RPS)],
                        out_hbm.at[c, pl.ds(s * RPS, RPS)])

    return deg


def _tc_l1(x, W1, degp):
    def body(x_ref, w_ref, d_ref, h_ref, hs_ref, dinv_ref):
        h = jnp.dot(x_ref[...], w_ref[...],
                    preferred_element_type=jnp.float32)
        h = jnp.pad(h, ((0, NPAD - N), (0, 0)))
        deg = d_ref[0, :, :1] + d_ref[1, :, :1] + 1.0
        dinv = lax.rsqrt(deg)
        h_ref[...] = h
        hs_ref[...] = h * dinv
        dinv_ref[...] = dinv

    return pl.pallas_call(
        body,
        out_shape=[
            jax.ShapeDtypeStruct((NPAD, HID), jnp.float32),
            jax.ShapeDtypeStruct((NPAD, HID), jnp.float32),
            jax.ShapeDtypeStruct((NPAD, 1), jnp.float32),
        ],
    )(x, W1, degp)


def _tc_relu(p, h1, dinv, b1):
    def body(p_ref, h1_ref, dinv_ref, b1_ref, z_ref, zs_ref):
        dinv = dinv_ref[...]
        z = dinv * (p_ref[0] + p_ref[1]) + (dinv * dinv) * h1_ref[...] \
            + b1_ref[...]
        z = jnp.maximum(z, 0.0)
        z_ref[...] = z
        zs_ref[...] = z * dinv

    return pl.pallas_call(
        body,
        out_shape=[
            jax.ShapeDtypeStruct((NPAD, HID), jnp.float32),
            jax.ShapeDtypeStruct((NPAD, HID), jnp.float32),
        ],
    )(p, h1, dinv, b1.reshape(1, HID))


def _tc_out(q, z1, dinv, W2, b2):
    def body(q_ref, z_ref, dinv_ref, w2_ref, b2_ref, out_ref):
        dinv = dinv_ref[:N]
        agg = dinv * (q_ref[0, :N] + q_ref[1, :N]) \
            + (dinv * dinv) * z_ref[:N]
        out_ref[...] = jnp.dot(agg, w2_ref[...],
                               preferred_element_type=jnp.float32) \
            + b2_ref[...]

    return pl.pallas_call(
        body,
        out_shape=jax.ShapeDtypeStruct((N, NCLS), jnp.float32),
    )(q, z1, dinv, W2, b2.reshape(1, NCLS))


def kernel(x, edge_index, W1, b1, W2, b2):
    src = edge_index[0]
    dst = edge_index[1]
    pad = E_PAD - E
    src_p = jnp.concatenate(
        [src, jnp.zeros((pad,), jnp.int32)]).reshape(NW * K, B)
    dst_p = jnp.concatenate(
        [dst, jnp.full((pad,), N, jnp.int32)]).reshape(NW * K, B)

    ones_deg = jnp.ones((B, DEGW), jnp.float32)
    zeros_deg = jnp.zeros((RPS, DEGW), jnp.float32)
    zeros_h = jnp.zeros((RPS, HID), jnp.float32)

    agg = _sc_agg()
    degp = _sc_deg()(dst_p, ones_deg, zeros_deg)
    h1, hs1, dinv = _tc_l1(x, W1, degp)
    p = agg(hs1, src_p, dst_p, zeros_h)
    z1, zs1 = _tc_relu(p, h1, dinv, b1)
    q = agg(zs1, src_p, dst_p, zeros_h)
    return _tc_out(q, z1, dinv, W2, b2)


# speedup vs baseline: 1.9489x; 1.0499x over previous
import functools

import jax
import jax.numpy as jnp
from jax import lax
from jax.experimental import pallas as pl
from jax.experimental.pallas import tpu as pltpu
from jax.experimental.pallas import tpu_sc as plsc

N = 10000
E = 320000
DIN = 128
HID = 16
NCLS = 40

NCORE = 2
NSUB = 16
NW = NCORE * NSUB
B = 128
K = 80
NCH = E // B
KLAST = NCH - 31 * K
E_PAD = NW * B * K
NPAD = 10112
RPS = NPAD // NSUB
DEGW = 16
NB = 8

_SC_PARAMS = pltpu.CompilerParams(use_tc_tiling_on_sc=False)


def _stage_edges(ei_hbm, src_v, dst_v, w):
    @pl.when(w < NW - 1)
    def _():
        pltpu.sync_copy(ei_hbm.at[0, pl.ds(w * K, K)], src_v)
        pltpu.sync_copy(ei_hbm.at[1, pl.ds(w * K, K)], dst_v)

    @pl.when(w == NW - 1)
    def _():
        pltpu.sync_copy(ei_hbm.at[0, pl.ds((NW - 1) * K, KLAST)],
                        src_v.at[pl.ds(0, KLAST)])
        pltpu.sync_copy(ei_hbm.at[1, pl.ds((NW - 1) * K, KLAST)],
                        dst_v.at[pl.ds(0, KLAST)])
        zero16 = jnp.zeros((16,), jnp.int32)
        sink16 = jnp.full((16,), N, jnp.int32)

        def fill(r, carry):
            for g in range(B // 16):
                src_v[r, pl.ds(g * 16, 16)] = zero16
                dst_v[r, pl.ds(g * 16, 16)] = sink16
            return carry

        lax.fori_loop(KLAST, K, fill, 0)


def _sc_agg():
    D = HID
    mesh = plsc.VectorSubcoreMesh(core_axis_name="c", subcore_axis_name="s")

    @functools.partial(
        pl.kernel,
        out_type=jax.ShapeDtypeStruct((NCORE, NPAD, D), jnp.float32),
        mesh=mesh,
        scratch_types=[
            pltpu.VMEM((K, B), jnp.int32),
            pltpu.VMEM((K, B), jnp.int32),
            pltpu.VMEM((NB, B, D), jnp.float32),
            pltpu.VMEM_SHARED((NPAD, D), jnp.float32),
            pltpu.VMEM_SHARED((NPAD, D), jnp.float32),
        ] + [pltpu.SemaphoreType.DMA] * (2 * NB),
        compiler_params=_SC_PARAMS,
    )
    def agg(hs_hbm, ei_hbm, zeros_hbm, out_hbm,
            src_v, dst_v, rows_v, acc_sh, tab_sh, *sems):
        gsem = sems[:NB]
        ssem = sems[NB:]
        c = lax.axis_index("c")
        s = lax.axis_index("s")
        w = s * NCORE + c
        _stage_edges(ei_hbm, src_v, dst_v, w)
        pltpu.sync_copy(zeros_hbm, acc_sh.at[pl.ds(s * RPS, RPS)])
        pltpu.sync_copy(hs_hbm.at[pl.ds(s * RPS, RPS)],
                        tab_sh.at[pl.ds(s * RPS, RPS)])
        plsc.subcore_barrier()

        pltpu.async_copy(tab_sh.at[src_v.at[0]], rows_v.at[0], gsem[0])

        def outer(t, carry):
            for b in range(NB):
                j = t * NB + b
                bn = (b + 1) % NB
                jn = j + 1
                pltpu.make_async_copy(
                    tab_sh.at[src_v.at[j]], rows_v.at[b], gsem[b]).wait()
                pltpu.async_copy(rows_v.at[b],
                                 acc_sh.at[dst_v.at[j]], ssem[b], add=True)

                @pl.when(jnp.logical_and(jn >= NB, jn < K))
                def _():
                    pltpu.make_async_copy(
                        rows_v.at[bn],
                        acc_sh.at[dst_v.at[jn - NB]], ssem[bn]).wait()

                @pl.when(jn < K)
                def _():
                    pltpu.async_copy(
                        tab_sh.at[src_v.at[jn]], rows_v.at[bn], gsem[bn])
            return carry

        lax.fori_loop(0, K // NB, outer, 0)
        for cch in range(K - NB, K):
            pltpu.make_async_copy(
                rows_v.at[cch % NB],
                acc_sh.at[dst_v.at[cch]], ssem[cch % NB]).wait()
        plsc.subcore_barrier()
        pltpu.sync_copy(acc_sh.at[pl.ds(s * RPS, RPS)],
                        out_hbm.at[c, pl.ds(s * RPS, RPS)])

    return agg


def _sc_deg():
    mesh = plsc.VectorSubcoreMesh(core_axis_name="c", subcore_axis_name="s")

    @functools.partial(
        pl.kernel,
        out_type=jax.ShapeDtypeStruct((NCORE, NPAD, DEGW), jnp.float32),
        mesh=mesh,
        scratch_types=[
            pltpu.VMEM((K, B), jnp.int32),
            pltpu.VMEM((K, B), jnp.int32),
            pltpu.VMEM((B, DEGW), jnp.float32),
            pltpu.VMEM_SHARED((NPAD, DEGW), jnp.float32),
        ] + [pltpu.SemaphoreType.DMA] * NB,
        compiler_params=_SC_PARAMS,
    )
    def deg(ei_hbm, ones_hbm, zeros_hbm, out_hbm, src_v, dst_v, ones_v,
            acc_sh, *sems):
        c = lax.axis_index("c")
        s = lax.axis_index("s")
        w = s * NCORE + c
        _stage_edges(ei_hbm, src_v, dst_v, w)
        pltpu.sync_copy(ones_hbm, ones_v)
        pltpu.sync_copy(zeros_hbm, acc_sh.at[pl.ds(s * RPS, RPS)])
        plsc.subcore_barrier()

        for j0 in range(NB):
            pltpu.async_copy(ones_v, acc_sh.at[dst_v.at[j0]], sems[j0],
                             add=True)

        def step(j, carry):
            for b in range(NB):
                pltpu.make_async_copy(
                    ones_v, acc_sh.at[dst_v.at[j * NB + b]], sems[b]).wait()

                @pl.when(j * NB + b + NB < K)
                def _():
                    pltpu.async_copy(
                        ones_v, acc_sh.at[dst_v.at[j * NB + b + NB]],
                        sems[b], add=True)
            return carry

        lax.fori_loop(0, K // NB, step, 0)
        plsc.subcore_barrier()
        pltpu.sync_copy(acc_sh.at[pl.ds(s * RPS, RPS)],
                        out_hbm.at[c, pl.ds(s * RPS, RPS)])

    return deg


def _tc_l1(x, W1, degp):
    def body(x_ref, w_ref, d_ref, h_ref, hs_ref, dinv_ref):
        h = jnp.dot(x_ref[...], w_ref[...],
                    preferred_element_type=jnp.float32)
        h = jnp.pad(h, ((0, NPAD - N), (0, 0)))
        deg = d_ref[0, :, :1] + d_ref[1, :, :1] + 1.0
        dinv = lax.rsqrt(deg)
        h_ref[...] = h
        hs_ref[...] = h * dinv
        dinv_ref[...] = dinv

    return pl.pallas_call(
        body,
        out_shape=[
            jax.ShapeDtypeStruct((NPAD, HID), jnp.float32),
            jax.ShapeDtypeStruct((NPAD, HID), jnp.float32),
            jax.ShapeDtypeStruct((NPAD, 1), jnp.float32),
        ],
    )(x, W1, degp)


def _tc_relu(p, h1, dinv, b1):
    def body(p_ref, h1_ref, dinv_ref, b1_ref, z_ref, zs_ref):
        dinv = dinv_ref[...]
        z = dinv * (p_ref[0] + p_ref[1]) + (dinv * dinv) * h1_ref[...] \
            + b1_ref[...]
        z = jnp.maximum(z, 0.0)
        z_ref[...] = z
        zs_ref[...] = z * dinv

    return pl.pallas_call(
        body,
        out_shape=[
            jax.ShapeDtypeStruct((NPAD, HID), jnp.float32),
            jax.ShapeDtypeStruct((NPAD, HID), jnp.float32),
        ],
    )(p, h1, dinv, b1.reshape(1, HID))


def _tc_out(q, z1, dinv, W2, b2):
    def body(q_ref, z_ref, dinv_ref, w2_ref, b2_ref, out_ref):
        dinv = dinv_ref[:N]
        agg = dinv * (q_ref[0, :N] + q_ref[1, :N]) \
            + (dinv * dinv) * z_ref[:N]
        out_ref[...] = jnp.dot(agg, w2_ref[...],
                               preferred_element_type=jnp.float32) \
            + b2_ref[...]

    return pl.pallas_call(
        body,
        out_shape=jax.ShapeDtypeStruct((N, NCLS), jnp.float32),
    )(q, z1, dinv, W2, b2.reshape(1, NCLS))


def kernel(x, edge_index, W1, b1, W2, b2):
    ei3 = edge_index.reshape(2, NCH, B)

    ones_deg = jnp.ones((B, DEGW), jnp.float32)
    zeros_deg = jnp.zeros((RPS, DEGW), jnp.float32)
    zeros_h = jnp.zeros((RPS, HID), jnp.float32)

    agg = _sc_agg()
    degp = _sc_deg()(ei3, ones_deg, zeros_deg)
    h1, hs1, dinv = _tc_l1(x, W1, degp)
    p = agg(hs1, ei3, zeros_h)
    z1, zs1 = _tc_relu(p, h1, dinv, b1)
    q = agg(zs1, ei3, zeros_h)
    return _tc_out(q, z1, dinv, W2, b2)


# speedup vs baseline: 2.0718x; 1.0630x over previous
import functools

import jax
import jax.numpy as jnp
from jax import lax
from jax.experimental import pallas as pl
from jax.experimental.pallas import tpu as pltpu
from jax.experimental.pallas import tpu_sc as plsc

N = 10000
E = 320000
DIN = 128
HID = 16
NCLS = 40

NCORE = 2
NSUB = 16
NW = NCORE * NSUB
B = 128
K = 80
NCH = E // B
KLAST = NCH - 31 * K
NPAD = 10112
RPS = NPAD // NSUB
DEGW = 16
NB = 8

_SC_PARAMS = pltpu.CompilerParams(use_tc_tiling_on_sc=False,
                                  needs_layout_passes=False)


def _stage_edges(ei_hbm, src_v, dst_v, w):
    @pl.when(w < NW - 1)
    def _():
        pltpu.sync_copy(ei_hbm.at[0, pl.ds(w * K, K)], src_v)
        pltpu.sync_copy(ei_hbm.at[1, pl.ds(w * K, K)], dst_v)

    @pl.when(w == NW - 1)
    def _():
        pltpu.sync_copy(ei_hbm.at[0, pl.ds((NW - 1) * K, KLAST)],
                        src_v.at[pl.ds(0, KLAST)])
        pltpu.sync_copy(ei_hbm.at[1, pl.ds((NW - 1) * K, KLAST)],
                        dst_v.at[pl.ds(0, KLAST)])
        zero16 = jnp.zeros((16,), jnp.int32)
        sink16 = jnp.full((16,), N, jnp.int32)

        def fill(r, carry):
            for g in range(B // 16):
                src_v[r, pl.ds(g * 16, 16)] = zero16
                dst_v[r, pl.ds(g * 16, 16)] = sink16
            return carry

        lax.fori_loop(KLAST, K, fill, 0)


def _dinv_vec(d):
    i = plsc.bitcast(d, jnp.int32)
    y = plsc.bitcast(jnp.int32(0x5F3759DF) - (i >> 1), jnp.float32)
    for _ in range(3):
        y = y * (1.5 - 0.5 * d * y * y)
    return y


def _ring_agg(tab_sh, acc_sh, src_v, dst_v, rows_v, gsem, ssem):
    pltpu.async_copy(tab_sh.at[src_v.at[0]], rows_v.at[0], gsem[0])

    def outer(t, carry):
        for b in range(NB):
            j = t * NB + b
            bn = (b + 1) % NB
            jn = j + 1
            pltpu.make_async_copy(
                tab_sh.at[src_v.at[j]], rows_v.at[b], gsem[b]).wait()
            pltpu.async_copy(rows_v.at[b],
                             acc_sh.at[dst_v.at[j]], ssem[b], add=True)

            @pl.when(jnp.logical_and(jn >= NB, jn < K))
            def _():
                pltpu.make_async_copy(
                    rows_v.at[bn],
                    acc_sh.at[dst_v.at[jn - NB]], ssem[bn]).wait()

            @pl.when(jn < K)
            def _():
                pltpu.async_copy(
                    tab_sh.at[src_v.at[jn]], rows_v.at[bn], gsem[bn])
        return carry

    lax.fori_loop(0, K // NB, outer, 0)
    for cch in range(K - NB, K):
        pltpu.make_async_copy(
            rows_v.at[cch % NB],
            acc_sh.at[dst_v.at[cch]], ssem[cch % NB]).wait()


def _sc_agg1():
    @functools.partial(
        pl.kernel,
        out_type=jax.ShapeDtypeStruct((NCORE, NPAD, HID), jnp.float32),
        mesh=plsc.VectorSubcoreMesh(core_axis_name="c",
                                    subcore_axis_name="s"),
        scratch_types=[
            pltpu.VMEM((K, B), jnp.int32),
            pltpu.VMEM((K, B), jnp.int32),
            pltpu.VMEM((NB, B, HID), jnp.float32),
            pltpu.VMEM((RPS, HID), jnp.float32),
            pltpu.VMEM((RPS, HID), jnp.float32),
            pltpu.VMEM((RPS, HID), jnp.float32),
            pltpu.VMEM_SHARED((NPAD, HID), jnp.float32),
            pltpu.VMEM_SHARED((NPAD, HID), jnp.float32),
        ] + [pltpu.SemaphoreType.DMA] * (2 * NB),
        compiler_params=_SC_PARAMS,
    )
    def agg1(h1_hbm, degp_hbm, ei_hbm, zeros_hbm, out_hbm,
             src_v, dst_v, rows_v, ha, da, pa, acc_sh, tab_sh, *sems):
        gsem = sems[:NB]
        ssem = sems[NB:]
        c = lax.axis_index("c")
        s = lax.axis_index("s")
        w = s * NCORE + c
        _stage_edges(ei_hbm, src_v, dst_v, w)
        rows = pl.ds(s * RPS, RPS)
        pltpu.sync_copy(zeros_hbm, acc_sh.at[rows])
        pltpu.sync_copy(h1_hbm.at[rows], ha)
        pltpu.sync_copy(degp_hbm.at[0, rows], da)
        pltpu.sync_copy(degp_hbm.at[1, rows], pa)

        def hs_row(r, carry):
            dv = _dinv_vec(da[r, :] + pa[r, :] + 1.0)
            ha[r, :] = ha[r, :] * dv
            return carry

        lax.fori_loop(0, RPS, hs_row, 0)
        pltpu.sync_copy(ha, tab_sh.at[rows])
        plsc.subcore_barrier()
        _ring_agg(tab_sh, acc_sh, src_v, dst_v, rows_v, gsem, ssem)
        plsc.subcore_barrier()
        pltpu.sync_copy(acc_sh.at[rows], out_hbm.at[c, rows])

    return agg1


def _sc_agg2():
    @functools.partial(
        pl.kernel,
        out_type=[
            jax.ShapeDtypeStruct((NCORE, NPAD, HID), jnp.float32),
            jax.ShapeDtypeStruct((NPAD, HID), jnp.float32),
            jax.ShapeDtypeStruct((NPAD, HID), jnp.float32),
        ],
        mesh=plsc.VectorSubcoreMesh(core_axis_name="c",
                                    subcore_axis_name="s"),
        scratch_types=[
            pltpu.VMEM((K, B), jnp.int32),
            pltpu.VMEM((K, B), jnp.int32),
            pltpu.VMEM((NB, B, HID), jnp.float32),
            pltpu.VMEM((RPS, HID), jnp.float32),
            pltpu.VMEM((RPS, HID), jnp.float32),
            pltpu.VMEM((RPS, HID), jnp.float32),
            pltpu.VMEM((RPS, HID), jnp.float32),
            pltpu.VMEM((16,), jnp.float32),
            pltpu.VMEM_SHARED((NPAD, HID), jnp.float32),
            pltpu.VMEM_SHARED((NPAD, HID), jnp.float32),
        ] + [pltpu.SemaphoreType.DMA] * (2 * NB),
        compiler_params=_SC_PARAMS,
    )
    def agg2(h1_hbm, degp_hbm, p_hbm, b1_hbm, ei_hbm, zeros_hbm,
             q_hbm, z1_hbm, dinv_hbm,
             src_v, dst_v, rows_v, ha, da, pa, pb, bb,
             acc_sh, tab_sh, *sems):
        gsem = sems[:NB]
        ssem = sems[NB:]
        c = lax.axis_index("c")
        s = lax.axis_index("s")
        w = s * NCORE + c
        _stage_edges(ei_hbm, src_v, dst_v, w)
        rows = pl.ds(s * RPS, RPS)
        pltpu.sync_copy(zeros_hbm, acc_sh.at[rows])
        pltpu.sync_copy(h1_hbm.at[rows], ha)
        pltpu.sync_copy(degp_hbm.at[0, rows], da)
        pltpu.sync_copy(degp_hbm.at[1, rows], pa)
        pltpu.sync_copy(b1_hbm, bb)

        def dinv_row(r, carry):
            da[r, :] = _dinv_vec(da[r, :] + pa[r, :] + 1.0)
            return carry

        lax.fori_loop(0, RPS, dinv_row, 0)
        pltpu.sync_copy(p_hbm.at[0, rows], pa)
        pltpu.sync_copy(p_hbm.at[1, rows], pb)
        b1v = bb[...]

        def z_row(r, carry):
            dv = da[r, :]
            z = dv * (pa[r, :] + pb[r, :]) + dv * dv * ha[r, :] + b1v
            z = jnp.maximum(z, 0.0)
            pa[r, :] = z
            ha[r, :] = z * dv
            return carry

        lax.fori_loop(0, RPS, z_row, 0)
        pltpu.sync_copy(ha, tab_sh.at[rows])

        @pl.when(c == 0)
        def _():
            pltpu.sync_copy(pa, z1_hbm.at[rows])
            pltpu.sync_copy(da, dinv_hbm.at[rows])

        plsc.subcore_barrier()
        _ring_agg(tab_sh, acc_sh, src_v, dst_v, rows_v, gsem, ssem)
        plsc.subcore_barrier()
        pltpu.sync_copy(acc_sh.at[rows], q_hbm.at[c, rows])

    return agg2


def _sc_deg():
    @functools.partial(
        pl.kernel,
        out_type=jax.ShapeDtypeStruct((NCORE, NPAD, DEGW), jnp.float32),
        mesh=plsc.VectorSubcoreMesh(core_axis_name="c",
                                    subcore_axis_name="s"),
        scratch_types=[
            pltpu.VMEM((K, B), jnp.int32),
            pltpu.VMEM((K, B), jnp.int32),
            pltpu.VMEM((B, DEGW), jnp.float32),
            pltpu.VMEM_SHARED((NPAD, DEGW), jnp.float32),
        ] + [pltpu.SemaphoreType.DMA] * NB,
        compiler_params=_SC_PARAMS,
    )
    def deg(ei_hbm, ones_hbm, zeros_hbm, out_hbm, src_v, dst_v, ones_v,
            acc_sh, *sems):
        c = lax.axis_index("c")
        s = lax.axis_index("s")
        w = s * NCORE + c
        _stage_edges(ei_hbm, src_v, dst_v, w)
        pltpu.sync_copy(ones_hbm, ones_v)
        pltpu.sync_copy(zeros_hbm, acc_sh.at[pl.ds(s * RPS, RPS)])
        plsc.subcore_barrier()

        for j0 in range(NB):
            pltpu.async_copy(ones_v, acc_sh.at[dst_v.at[j0]], sems[j0],
                             add=True)

        def step(j, carry):
            for b in range(NB):
                pltpu.make_async_copy(
                    ones_v, acc_sh.at[dst_v.at[j * NB + b]], sems[b]).wait()

                @pl.when(j * NB + b + NB < K)
                def _():
                    pltpu.async_copy(
                        ones_v, acc_sh.at[dst_v.at[j * NB + b + NB]],
                        sems[b], add=True)
            return carry

        lax.fori_loop(0, K // NB, step, 0)
        plsc.subcore_barrier()
        pltpu.sync_copy(acc_sh.at[pl.ds(s * RPS, RPS)],
                        out_hbm.at[c, pl.ds(s * RPS, RPS)])

    return deg


def _tc_mm1(x, W1):
    def body(x_ref, w_ref, h_ref):
        h = jnp.dot(x_ref[...], w_ref[...],
                    preferred_element_type=jnp.float32)
        h_ref[...] = jnp.pad(h, ((0, NPAD - N), (0, 0)))

    return pl.pallas_call(
        body,
        out_shape=jax.ShapeDtypeStruct((NPAD, HID), jnp.float32),
    )(x, W1)


def _tc_out(q, z1, dinv, W2, b2):
    def body(q_ref, z_ref, dinv_ref, w2_ref, b2_ref, out_ref):
        dv = dinv_ref[:N, :1]
        agg = dv * (q_ref[0, :N] + q_ref[1, :N]) \
            + (dv * dv) * z_ref[:N]
        out_ref[...] = jnp.dot(agg, w2_ref[...],
                               preferred_element_type=jnp.float32) \
            + b2_ref[...]

    return pl.pallas_call(
        body,
        out_shape=jax.ShapeDtypeStruct((N, NCLS), jnp.float32),
    )(q, z1, dinv, W2, b2.reshape(1, NCLS))


def kernel(x, edge_index, W1, b1, W2, b2):
    ei3 = edge_index.reshape(2, NCH, B)

    ones_deg = jnp.ones((B, DEGW), jnp.float32)
    zeros_deg = jnp.zeros((RPS, DEGW), jnp.float32)
    zeros_h = jnp.zeros((RPS, HID), jnp.float32)

    h1 = _tc_mm1(x, W1)
    degp = _sc_deg()(ei3, ones_deg, zeros_deg)
    p = _sc_agg1()(h1, degp, ei3, zeros_h)
    q, z1, dinv = _sc_agg2()(h1, degp, p, b1, ei3, zeros_h)
    return _tc_out(q, z1, dinv, W2, b2)


# speedup vs baseline: 2.2822x; 1.1016x over previous
import functools

import jax
import jax.numpy as jnp
from jax import lax
from jax.experimental import pallas as pl
from jax.experimental.pallas import tpu as pltpu
from jax.experimental.pallas import tpu_sc as plsc

N = 10000
E = 320000
DIN = 128
HID = 16
NCLS = 40

NCORE = 2
NSUB = 16
NW = NCORE * NSUB
B = 128
K = 80
NCH = E // B
KLAST = NCH - 31 * K
NPAD = 10112
RPS = NPAD // NSUB
DEGW = 16
NB = 8

_SC_PARAMS = pltpu.CompilerParams(use_tc_tiling_on_sc=False,
                                  needs_layout_passes=False)


def _stage_edges(ei_hbm, src_v, dst_v, w):
    @pl.when(w < NW - 1)
    def _():
        pltpu.sync_copy(ei_hbm.at[0, pl.ds(w * K, K)], src_v)
        pltpu.sync_copy(ei_hbm.at[1, pl.ds(w * K, K)], dst_v)

    @pl.when(w == NW - 1)
    def _():
        pltpu.sync_copy(ei_hbm.at[0, pl.ds((NW - 1) * K, KLAST)],
                        src_v.at[pl.ds(0, KLAST)])
        pltpu.sync_copy(ei_hbm.at[1, pl.ds((NW - 1) * K, KLAST)],
                        dst_v.at[pl.ds(0, KLAST)])
        zero16 = jnp.zeros((16,), jnp.int32)
        sink16 = jnp.full((16,), N, jnp.int32)

        def fill(r, carry):
            for g in range(B // 16):
                src_v[r, pl.ds(g * 16, 16)] = zero16
                dst_v[r, pl.ds(g * 16, 16)] = sink16
            return carry

        lax.fori_loop(KLAST, K, fill, 0)


def _dinv_vec(d):
    i = plsc.bitcast(d, jnp.int32)
    y = plsc.bitcast(jnp.int32(0x5F3759DF) - (i >> 1), jnp.float32)
    for _ in range(3):
        y = y * (1.5 - 0.5 * d * y * y)
    return y


def _ring_agg(tab_sh, acc_sh, src_v, dst_v, rows_v, gsem, ssem):
    pltpu.async_copy(tab_sh.at[src_v.at[0]], rows_v.at[0], gsem[0])

    def outer(t, carry):
        for b in range(NB):
            j = t * NB + b
            bn = (b + 1) % NB
            jn = j + 1
            pltpu.make_async_copy(
                tab_sh.at[src_v.at[j]], rows_v.at[b], gsem[b]).wait()
            pltpu.async_copy(rows_v.at[b],
                             acc_sh.at[dst_v.at[j]], ssem[b], add=True)

            @pl.when(jnp.logical_and(jn >= NB, jn < K))
            def _():
                pltpu.make_async_copy(
                    rows_v.at[bn],
                    acc_sh.at[dst_v.at[jn - NB]], ssem[bn]).wait()

            @pl.when(jn < K)
            def _():
                pltpu.async_copy(
                    tab_sh.at[src_v.at[jn]], rows_v.at[bn], gsem[bn])
        return carry

    lax.fori_loop(0, K // NB, outer, 0)
    for cch in range(K - NB, K):
        pltpu.make_async_copy(
            rows_v.at[cch % NB],
            acc_sh.at[dst_v.at[cch]], ssem[cch % NB]).wait()


def _sc_agg1():
    @functools.partial(
        pl.kernel,
        out_type=jax.ShapeDtypeStruct((NCORE, NPAD, HID), jnp.float32),
        mesh=plsc.VectorSubcoreMesh(core_axis_name="c",
                                    subcore_axis_name="s"),
        scratch_types=[
            pltpu.VMEM((K, B), jnp.int32),
            pltpu.VMEM((K, B), jnp.int32),
            pltpu.VMEM((NB, B, HID), jnp.float32),
            pltpu.VMEM((RPS, HID), jnp.float32),
            pltpu.VMEM((RPS, HID), jnp.float32),
            pltpu.VMEM((RPS, HID), jnp.float32),
            pltpu.VMEM_SHARED((NPAD, HID), jnp.float32),
            pltpu.VMEM_SHARED((NPAD, HID), jnp.float32),
        ] + [pltpu.SemaphoreType.DMA] * (2 * NB),
        compiler_params=_SC_PARAMS,
    )
    def agg1(h1_hbm, degp_hbm, ei_hbm, zeros_hbm, out_hbm,
             src_v, dst_v, rows_v, ha, da, pa, acc_sh, tab_sh, *sems):
        gsem = sems[:NB]
        ssem = sems[NB:]
        c = lax.axis_index("c")
        s = lax.axis_index("s")
        w = s * NCORE + c
        _stage_edges(ei_hbm, src_v, dst_v, w)
        rows = pl.ds(s * RPS, RPS)
        pltpu.sync_copy(zeros_hbm, acc_sh.at[rows])
        pltpu.sync_copy(h1_hbm.at[rows], ha)
        pltpu.sync_copy(degp_hbm.at[0, rows], da)
        pltpu.sync_copy(degp_hbm.at[1, rows], pa)

        def hs_row(t, carry):
            for u in range(4):
                r = t * 4 + u
                dv = _dinv_vec(da[r, :] + pa[r, :] + 1.0)
                ha[r, :] = ha[r, :] * dv
            return carry

        lax.fori_loop(0, RPS // 4, hs_row, 0)
        pltpu.sync_copy(ha, tab_sh.at[rows])
        plsc.subcore_barrier()
        _ring_agg(tab_sh, acc_sh, src_v, dst_v, rows_v, gsem, ssem)
        plsc.subcore_barrier()
        pltpu.sync_copy(acc_sh.at[rows], out_hbm.at[c, rows])

    return agg1


def _sc_agg2():
    @functools.partial(
        pl.kernel,
        out_type=[
            jax.ShapeDtypeStruct((NCORE, NPAD, HID), jnp.float32),
            jax.ShapeDtypeStruct((NPAD, HID), jnp.float32),
            jax.ShapeDtypeStruct((NPAD, HID), jnp.float32),
        ],
        mesh=plsc.VectorSubcoreMesh(core_axis_name="c",
                                    subcore_axis_name="s"),
        scratch_types=[
            pltpu.VMEM((K, B), jnp.int32),
            pltpu.VMEM((K, B), jnp.int32),
            pltpu.VMEM((NB, B, HID), jnp.float32),
            pltpu.VMEM((RPS, HID), jnp.float32),
            pltpu.VMEM((RPS, HID), jnp.float32),
            pltpu.VMEM((RPS, HID), jnp.float32),
            pltpu.VMEM((RPS, HID), jnp.float32),
            pltpu.VMEM((RPS, HID), jnp.float32),
            pltpu.VMEM((16,), jnp.float32),
            pltpu.VMEM_SHARED((NPAD, HID), jnp.float32),
            pltpu.VMEM_SHARED((NPAD, HID), jnp.float32),
        ] + [pltpu.SemaphoreType.DMA] * (2 * NB),
        compiler_params=_SC_PARAMS,
    )
    def agg2(h1_hbm, degp_hbm, p_hbm, b1_hbm, ei_hbm, zeros_hbm,
             q_hbm, z1_hbm, dinv_hbm,
             src_v, dst_v, rows_v, ha, da, db, pa, pb, bb,
             acc_sh, tab_sh, *sems):
        gsem = sems[:NB]
        ssem = sems[NB:]
        c = lax.axis_index("c")
        s = lax.axis_index("s")
        w = s * NCORE + c
        _stage_edges(ei_hbm, src_v, dst_v, w)
        rows = pl.ds(s * RPS, RPS)
        pltpu.sync_copy(zeros_hbm, acc_sh.at[rows])
        pltpu.sync_copy(h1_hbm.at[rows], ha)
        pltpu.sync_copy(degp_hbm.at[0, rows], da)
        pltpu.sync_copy(degp_hbm.at[1, rows], db)
        pltpu.sync_copy(p_hbm.at[0, rows], pa)
        pltpu.sync_copy(p_hbm.at[1, rows], pb)
        pltpu.sync_copy(b1_hbm, bb)
        b1v = bb[...]

        def z_row(t, carry):
            for u in range(4):
                r = t * 4 + u
                dv = _dinv_vec(da[r, :] + db[r, :] + 1.0)
                z = dv * (pa[r, :] + pb[r, :]) + dv * dv * ha[r, :] + b1v
                z = jnp.maximum(z, 0.0)
                pa[r, :] = z
                ha[r, :] = z * dv
                da[r, :] = dv
            return carry

        lax.fori_loop(0, RPS // 4, z_row, 0)
        pltpu.sync_copy(ha, tab_sh.at[rows])

        @pl.when(c == 0)
        def _():
            pltpu.sync_copy(pa, z1_hbm.at[rows])
            pltpu.sync_copy(da, dinv_hbm.at[rows])

        plsc.subcore_barrier()
        _ring_agg(tab_sh, acc_sh, src_v, dst_v, rows_v, gsem, ssem)
        plsc.subcore_barrier()
        pltpu.sync_copy(acc_sh.at[rows], q_hbm.at[c, rows])

    return agg2


def _sc_deg():
    @functools.partial(
        pl.kernel,
        out_type=jax.ShapeDtypeStruct((NCORE, NPAD, DEGW), jnp.float32),
        mesh=plsc.VectorSubcoreMesh(core_axis_name="c",
                                    subcore_axis_name="s"),
        scratch_types=[
            pltpu.VMEM((K, B), jnp.int32),
            pltpu.VMEM((K, B), jnp.int32),
            pltpu.VMEM((B, DEGW), jnp.float32),
            pltpu.VMEM_SHARED((NPAD, DEGW), jnp.float32),
        ] + [pltpu.SemaphoreType.DMA] * NB,
        compiler_params=_SC_PARAMS,
    )
    def deg(ei_hbm, ones_hbm, zeros_hbm, out_hbm, src_v, dst_v, ones_v,
            acc_sh, *sems):
        c = lax.axis_index("c")
        s = lax.axis_index("s")
        w = s * NCORE + c
        _stage_edges(ei_hbm, src_v, dst_v, w)
        pltpu.sync_copy(ones_hbm, ones_v)
        pltpu.sync_copy(zeros_hbm, acc_sh.at[pl.ds(s * RPS, RPS)])
        plsc.subcore_barrier()

        for j0 in range(NB):
            pltpu.async_copy(ones_v, acc_sh.at[dst_v.at[j0]], sems[j0],
                             add=True)

        def step(j, carry):
            for b in range(NB):
                pltpu.make_async_copy(
                    ones_v, acc_sh.at[dst_v.at[j * NB + b]], sems[b]).wait()

                @pl.when(j * NB + b + NB < K)
                def _():
                    pltpu.async_copy(
                        ones_v, acc_sh.at[dst_v.at[j * NB + b + NB]],
                        sems[b], add=True)
            return carry

        lax.fori_loop(0, K // NB, step, 0)
        plsc.subcore_barrier()
        pltpu.sync_copy(acc_sh.at[pl.ds(s * RPS, RPS)],
                        out_hbm.at[c, pl.ds(s * RPS, RPS)])

    return deg


def _tc_mm1(x, W1):
    def body(x_ref, w_ref, h_ref):
        h = jnp.dot(x_ref[...], w_ref[...],
                    preferred_element_type=jnp.float32)
        h_ref[...] = jnp.pad(h, ((0, NPAD - N), (0, 0)))

    return pl.pallas_call(
        body,
        out_shape=jax.ShapeDtypeStruct((NPAD, HID), jnp.float32),
    )(x, W1)


def _tc_out(q, z1, dinv, W2, b2):
    def body(q_ref, z_ref, dinv_ref, w2_ref, b2_ref, out_ref):
        dv = dinv_ref[:N, :1]
        agg = dv * (q_ref[0, :N] + q_ref[1, :N]) \
            + (dv * dv) * z_ref[:N]
        out_ref[...] = jnp.dot(agg, w2_ref[...],
                               preferred_element_type=jnp.float32) \
            + b2_ref[...]

    return pl.pallas_call(
        body,
        out_shape=jax.ShapeDtypeStruct((N, NCLS), jnp.float32),
    )(q, z1, dinv, W2, b2.reshape(1, NCLS))


def kernel(x, edge_index, W1, b1, W2, b2):
    ei3 = edge_index.reshape(2, NCH, B)

    ones_deg = jnp.ones((B, DEGW), jnp.float32)
    zeros_deg = jnp.zeros((RPS, DEGW), jnp.float32)
    zeros_h = jnp.zeros((RPS, HID), jnp.float32)

    h1 = _tc_mm1(x, W1)
    degp = _sc_deg()(ei3, ones_deg, zeros_deg)
    p = _sc_agg1()(h1, degp, ei3, zeros_h)
    q, z1, dinv = _sc_agg2()(h1, degp, p, b1, ei3, zeros_h)
    return _tc_out(q, z1, dinv, W2, b2)
